# speedup vs baseline: 1.0134x; 1.0134x over previous
import functools
import jax
import jax.numpy as jnp
from jax.experimental import pallas as pl

N = 100000
E = 6400000
F_IN = 4
HID = 8
HEADS = 2
CLS = 3
SLOPE = 0.2


def _gat_layer(h, a_src, a_dst, src, dst, n):
    g = jax.nn.leaky_relu(jnp.max(a_src) + jnp.max(a_dst), SLOPE)
    e = jax.nn.leaky_relu(a_src[src] + a_dst[dst], SLOPE) - g
    w = jnp.exp(e)
    h1 = jnp.concatenate([h, jnp.ones(h.shape[:-1] + (1,), h.dtype)], axis=-1)
    msg = h1[src] * w[:, :, None]
    acc = jax.ops.segment_sum(msg, dst, num_segments=n)
    w_self = jnp.exp(jax.nn.leaky_relu(a_src + a_dst, SLOPE) - g)
    acc = acc + h1 * w_self[:, :, None]
    num = acc[..., :-1]
    den = acc[..., -1:]
    return num / (den + 1e-16)


def _finish_kernel(num_ref, den_ref, o_ref):
    num = num_ref[...]
    den = den_ref[...]
    h = num / (den + 1e-16)
    m = jnp.max(h, axis=-1, keepdims=True)
    ex = jnp.exp(h - m)
    o_ref[...] = ex / jnp.sum(ex, axis=-1, keepdims=True)


def kernel(x, edge_index, W1, att_src1, att_dst1, W2, att_src2, att_dst2):
    src = edge_index[0]
    dst = edge_index[1]

    h = (x @ W1).reshape(N, HEADS, HID)
    a_src = (h * att_src1).sum(-1)
    a_dst = (h * att_dst1).sum(-1)
    out1 = _gat_layer(h, a_src, a_dst, src, dst, N).reshape(N, HEADS * HID)
    out1 = jax.nn.relu(out1)

    h2 = (out1 @ W2).reshape(N, 1, CLS)
    a_src2 = (h2 * att_src2).sum(-1)
    a_dst2 = (h2 * att_dst2).sum(-1)
    g2 = jax.nn.leaky_relu(jnp.max(a_src2) + jnp.max(a_dst2), SLOPE)
    e2 = jax.nn.leaky_relu(a_src2[src] + a_dst2[dst], SLOPE) - g2
    w2 = jnp.exp(e2)
    h2f = h2.reshape(N, CLS)
    msg2 = jnp.concatenate([h2f[src] * w2, w2], axis=-1)
    acc2 = jax.ops.segment_sum(msg2, dst, num_segments=N)
    w2s = jnp.exp(jax.nn.leaky_relu(a_src2 + a_dst2, SLOPE) - g2)
    acc2 = acc2 + jnp.concatenate([h2f * w2s, w2s], axis=-1)

    num2 = acc2[:, :CLS]
    den2 = acc2[:, CLS:]

    B = 8192
    npad = ((N + B - 1) // B) * B
    num2 = jnp.pad(num2, ((0, npad - N), (0, 0)))
    den2 = jnp.pad(den2, ((0, npad - N), (0, 0)), constant_values=1.0)
    out = pl.pallas_call(
        _finish_kernel,
        grid=(npad // B,),
        in_specs=[
            pl.BlockSpec((B, CLS), lambda i: (i, 0)),
            pl.BlockSpec((B, 1), lambda i: (i, 0)),
        ],
        out_specs=pl.BlockSpec((B, CLS), lambda i: (i, 0)),
        out_shape=jax.ShapeDtypeStruct((npad, CLS), jnp.float32),
    )(num2, den2)
    return out[:N]


# speedup vs baseline: 77.8563x; 76.8302x over previous
import functools
import jax
import jax.numpy as jnp
from jax import lax
from jax.experimental import pallas as pl
from jax.experimental.pallas import tpu as pltpu
from jax.experimental.pallas import tpu_sc as plsc

N = 100000
E = 6400000
F_IN = 4
HID = 8
HEADS = 2
CLS = 3
SLOPE = 0.2

NPAD = 102400
SLICE = NPAD // 16
B = 1024
K = 128
NCHUNK = E // K
L1_CHUNKS = NCHUNK // 16

_mesh = plsc.VectorSubcoreMesh(
    core_axis_name="c", subcore_axis_name="s", num_cores=2, num_subcores=16)

_sc_params = pltpu.CompilerParams(
    needs_layout_passes=False, use_tc_tiling_on_sc=False)


def _lrelu(x):
    return jnp.maximum(x, x * SLOPE)


def _stage_a_body(x_ref, w1_ref, as1_ref, ad1_ref,
                  table_ref, asrc_o_ref, adst_o_ref):
    h = jnp.dot(x_ref[...], w1_ref[...], preferred_element_type=jnp.float32)
    hs = h * as1_ref[...]
    hd = h * ad1_ref[...]
    z6 = jnp.zeros((B, 6), jnp.float32)
    for s in range(2):
        a_src = jnp.sum(hs[:, 8 * s:8 * s + 8], axis=1, keepdims=True)
        a_dst = jnp.sum(hd[:, 8 * s:8 * s + 8], axis=1, keepdims=True)
        table_ref[s] = jnp.concatenate(
            [h[:, 8 * s:8 * s + 8], a_src, a_dst, z6], axis=1)
        asrc_o_ref[:, s:s + 1] = a_src
        adst_o_ref[:, s:s + 1] = a_dst


def _stage_a(xpad, W1, as1, ad1):
    return pl.pallas_call(
        _stage_a_body,
        grid=(NPAD // B,),
        in_specs=[
            pl.BlockSpec((B, F_IN), lambda i: (i, 0)),
            pl.BlockSpec((F_IN, 16), lambda i: (0, 0)),
            pl.BlockSpec((1, 16), lambda i: (0, 0)),
            pl.BlockSpec((1, 16), lambda i: (0, 0)),
        ],
        out_specs=[
            pl.BlockSpec((2, B, 16), lambda i: (0, i, 0)),
            pl.BlockSpec((B, 2), lambda i: (i, 0)),
            pl.BlockSpec((B, 2), lambda i: (i, 0)),
        ],
        out_shape=[
            jax.ShapeDtypeStruct((2, NPAD, 16), jnp.float32),
            jax.ShapeDtypeStruct((NPAD, 2), jnp.float32),
            jax.ShapeDtypeStruct((NPAD, 2), jnp.float32),
        ],
    )(xpad, W1, as1, ad1)


def _sc_l1_body(ei, tab, zeros, gtab, acc_out,
                acc_sh, srcb, dstb, dstadj, rows, rowsd, msg, g16, sem):
    c = lax.axis_index("c")
    s = lax.axis_index("s")
    pltpu.sync_copy(gtab.at[c], g16)
    pltpu.sync_copy(zeros.at[pl.ds(0, K), :], msg)
    pltpu.sync_copy(zeros.at[pl.ds(s * SLICE, SLICE), :],
                    acc_sh.at[pl.ds(s * SLICE, SLICE), :])
    plsc.subcore_barrier()

    iota = lax.iota(jnp.int32, 16)
    g = g16[...]
    coff = jnp.full((16,), 0, jnp.int32) + c * NPAD
    col8 = jnp.full((16,), 8, jnp.int32)
    col9 = jnp.full((16,), 9, jnp.int32)

    @pl.loop(0, L1_CHUNKS)
    def _chunk(i):
        base = (s + 16 * i) * K
        pltpu.sync_copy(ei.at[0, pl.ds(base, K)], srcb)
        pltpu.sync_copy(ei.at[1, pl.ds(base, K)], dstb)
        for gk in range(8):
            srcb[pl.ds(gk * 16, 16)] = srcb[pl.ds(gk * 16, 16)] + coff
            dstadj[pl.ds(gk * 16, 16)] = dstb[pl.ds(gk * 16, 16)] + coff
        d1 = pltpu.async_copy(tab.at[srcb], rows, sem)
        d2 = pltpu.async_copy(tab.at[dstadj], rowsd, sem)
        d1.wait()
        d2.wait()
        for gk in range(8):
            r = iota + gk * 16
            ad = plsc.load_gather(rowsd, [r, col9])
            asr = plsc.load_gather(rows, [r, col8])
            w = jnp.exp(_lrelu(asr + ad) - g)
            for ch in range(8):
                cv = jnp.full((16,), ch, jnp.int32)
                col = plsc.load_gather(rows, [r, cv])
                plsc.store_scatter(msg, [r, cv], col * w)
            plsc.store_scatter(msg, [r, col8], w)
        pltpu.sync_copy(msg, acc_sh.at[dstb], add=True)

    plsc.subcore_barrier()
    pltpu.sync_copy(acc_sh.at[pl.ds(s * SLICE, SLICE), :],
                    acc_out.at[c, pl.ds(s * SLICE, SLICE), :])


_sc_l1 = functools.partial(
    pl.kernel,
    out_type=jax.ShapeDtypeStruct((2, NPAD, 16), jnp.float32),
    mesh=_mesh,
    compiler_params=_sc_params,
    scratch_types=[
        pltpu.VMEM_SHARED((NPAD, 16), jnp.float32),
        pltpu.VMEM((K,), jnp.int32),
        pltpu.VMEM((K,), jnp.int32),
        pltpu.VMEM((K,), jnp.int32),
        pltpu.VMEM((K, 16), jnp.float32),
        pltpu.VMEM((K, 16), jnp.float32),
        pltpu.VMEM((K, 16), jnp.float32),
        pltpu.VMEM((16,), jnp.float32),
        pltpu.SemaphoreType.DMA,
    ],
)(_sc_l1_body)


def _stage_b_body(acc_ref, tab_ref, asrc_ref, adst_ref, g1_ref,
                  w2_ref, as2_ref, ad2_ref,
                  table2_ref, asrc2_o_ref, adst2_o_ref):
    wself = jnp.exp(_lrelu(asrc_ref[...] + adst_ref[...]) - g1_ref[...])
    outs = []
    for s in range(2):
        h_s = tab_ref[s, :, 0:8]
        ws = wself[:, s:s + 1]
        num = acc_ref[s, :, 0:8] + ws * h_s
        den = acc_ref[s, :, 8:9] + ws
        outs.append(jnp.maximum(num / den, 0.0))
    out1 = jnp.concatenate(outs, axis=1)
    h2 = jnp.dot(out1, w2_ref[...], preferred_element_type=jnp.float32)
    asrc2 = jnp.sum(h2 * as2_ref[...], axis=1, keepdims=True)
    adst2 = jnp.sum(h2 * ad2_ref[...], axis=1, keepdims=True)
    table2_ref[...] = jnp.concatenate(
        [h2, asrc2, adst2, jnp.zeros((B, 11), jnp.float32)], axis=1)
    asrc2_o_ref[...] = asrc2
    adst2_o_ref[...] = adst2


def _stage_b(acc1, table1, asrc1, adst1, g1, W2, as2, ad2):
    return pl.pallas_call(
        _stage_b_body,
        grid=(NPAD // B,),
        in_specs=[
            pl.BlockSpec((2, B, 16), lambda i: (0, i, 0)),
            pl.BlockSpec((2, B, 16), lambda i: (0, i, 0)),
            pl.BlockSpec((B, 2), lambda i: (i, 0)),
            pl.BlockSpec((B, 2), lambda i: (i, 0)),
            pl.BlockSpec((1, 2), lambda i: (0, 0)),
            pl.BlockSpec((16, CLS), lambda i: (0, 0)),
            pl.BlockSpec((1, CLS), lambda i: (0, 0)),
            pl.BlockSpec((1, CLS), lambda i: (0, 0)),
        ],
        out_specs=[
            pl.BlockSpec((B, 16), lambda i: (i, 0)),
            pl.BlockSpec((B, 1), lambda i: (i, 0)),
            pl.BlockSpec((B, 1), lambda i: (i, 0)),
        ],
        out_shape=[
            jax.ShapeDtypeStruct((NPAD, 16), jnp.float32),
            jax.ShapeDtypeStruct((NPAD, 1), jnp.float32),
            jax.ShapeDtypeStruct((NPAD, 1), jnp.float32),
        ],
    )(acc1, table1, asrc1, adst1, g1, W2, as2, ad2)


def _sc_l2_body(ei, tab, zeros, gtab, acc_out,
                acc_sh, srcb, dstb, rows, rowsd, msg, g16, sem):
    c = lax.axis_index("c")
    s = lax.axis_index("s")
    wid = c * 16 + s
    pltpu.sync_copy(gtab.at[c], g16)
    pltpu.sync_copy(zeros.at[pl.ds(0, K), :], msg)
    pltpu.sync_copy(zeros.at[pl.ds(s * SLICE, SLICE), :],
                    acc_sh.at[pl.ds(s * SLICE, SLICE), :])
    plsc.subcore_barrier()

    iota = lax.iota(jnp.int32, 16)
    g = g16[...]
    col3 = jnp.full((16,), 3, jnp.int32)
    col4 = jnp.full((16,), 4, jnp.int32)
    nchunks = jnp.where(wid < 16, (NCHUNK + 31) // 32, NCHUNK // 32)

    @pl.loop(0, nchunks)
    def _chunk(i):
        base = (wid + 32 * i) * K
        pltpu.sync_copy(ei.at[0, pl.ds(base, K)], srcb)
        pltpu.sync_copy(ei.at[1, pl.ds(base, K)], dstb)
        d1 = pltpu.async_copy(tab.at[srcb], rows, sem)
        d2 = pltpu.async_copy(tab.at[dstb], rowsd, sem)
        d1.wait()
        d2.wait()
        for gk in range(8):
            r = iota + gk * 16
            ad = plsc.load_gather(rowsd, [r, col4])
            asr = plsc.load_gather(rows, [r, col3])
            w = jnp.exp(_lrelu(asr + ad) - g)
            for ch in range(CLS):
                cv = jnp.full((16,), ch, jnp.int32)
                col = plsc.load_gather(rows, [r, cv])
                plsc.store_scatter(msg, [r, cv], col * w)
            plsc.store_scatter(msg, [r, col3], w)
        pltpu.sync_copy(msg, acc_sh.at[dstb], add=True)

    plsc.subcore_barrier()
    pltpu.sync_copy(acc_sh.at[pl.ds(s * SLICE, SLICE), :],
                    acc_out.at[c, pl.ds(s * SLICE, SLICE), :])


_sc_l2 = functools.partial(
    pl.kernel,
    out_type=jax.ShapeDtypeStruct((2, NPAD, 16), jnp.float32),
    mesh=_mesh,
    compiler_params=_sc_params,
    scratch_types=[
        pltpu.VMEM_SHARED((NPAD, 16), jnp.float32),
        pltpu.VMEM((K,), jnp.int32),
        pltpu.VMEM((K,), jnp.int32),
        pltpu.VMEM((K, 16), jnp.float32),
        pltpu.VMEM((K, 16), jnp.float32),
        pltpu.VMEM((K, 16), jnp.float32),
        pltpu.VMEM((16,), jnp.float32),
        pltpu.SemaphoreType.DMA,
    ],
)(_sc_l2_body)


def _stage_c_body(acc_ref, tab2_ref, g2_ref, o_ref):
    h2 = tab2_ref[:, 0:CLS]
    asrc2 = tab2_ref[:, CLS:CLS + 1]
    adst2 = tab2_ref[:, CLS + 1:CLS + 2]
    wself = jnp.exp(_lrelu(asrc2 + adst2) - g2_ref[...])
    num = acc_ref[0, :, 0:CLS] + acc_ref[1, :, 0:CLS] + wself * h2
    den = acc_ref[0, :, CLS:CLS + 1] + acc_ref[1, :, CLS:CLS + 1] + wself
    h = num / den
    m = jnp.max(h, axis=-1, keepdims=True)
    ex = jnp.exp(h - m)
    o_ref[...] = ex / jnp.sum(ex, axis=-1, keepdims=True)


def _stage_c(acc2, table2, g2):
    return pl.pallas_call(
        _stage_c_body,
        grid=(NPAD // B,),
        in_specs=[
            pl.BlockSpec((2, B, 16), lambda i: (0, i, 0)),
            pl.BlockSpec((B, 16), lambda i: (i, 0)),
            pl.BlockSpec((1, 1), lambda i: (0, 0)),
        ],
        out_specs=pl.BlockSpec((B, CLS), lambda i: (i, 0)),
        out_shape=jax.ShapeDtypeStruct((NPAD, CLS), jnp.float32),
    )(acc2, table2, g2)


def kernel(x, edge_index, W1, att_src1, att_dst1, W2, att_src2, att_dst2):
    xpad = jnp.pad(x, ((0, NPAD - N), (0, 0)))
    as1 = att_src1.reshape(1, 16)
    ad1 = att_dst1.reshape(1, 16)
    table1, asrc1, adst1 = _stage_a(xpad, W1, as1, ad1)

    g1 = _lrelu(jnp.max(asrc1, axis=0) + jnp.max(adst1, axis=0))
    gtab1 = jnp.tile(g1[:, None], (1, 16))
    zeros16 = jnp.zeros((NPAD, 16), jnp.float32)

    acc1 = _sc_l1(edge_index, table1.reshape(2 * NPAD, 16), zeros16, gtab1)

    table2, asrc2, adst2 = _stage_b(
        acc1, table1, asrc1, adst1, g1.reshape(1, 2),
        W2, att_src2.reshape(1, CLS), att_dst2.reshape(1, CLS))

    g2 = _lrelu(jnp.max(asrc2) + jnp.max(adst2))
    gtab2 = jnp.full((2, 16), g2, jnp.float32)

    acc2 = _sc_l2(edge_index, table2, zeros16, gtab2)

    out = _stage_c(acc2, table2, g2.reshape(1, 1))
    return out[:N]


# speedup vs baseline: 190.2232x; 2.4433x over previous
import functools
import jax
import jax.numpy as jnp
from jax import lax
from jax.experimental import pallas as pl
from jax.experimental.pallas import tpu as pltpu
from jax.experimental.pallas import tpu_sc as plsc

N = 100000
E = 6400000
F_IN = 4
HID = 8
HEADS = 2
CLS = 3
SLOPE = 0.2

NPAD = 102400
SLICE = NPAD // 16
B = 1024
K = 128
NCHUNK = E // K
L1_CHUNKS = NCHUNK // 16

_mesh = plsc.VectorSubcoreMesh(
    core_axis_name="c", subcore_axis_name="s", num_cores=2, num_subcores=16)

_sc_params = pltpu.CompilerParams(
    needs_layout_passes=False, use_tc_tiling_on_sc=False)


def _lrelu(x):
    return jnp.maximum(x, x * SLOPE)


def _stage_a_body(x_ref, w1_ref, as1_ref, ad1_ref,
                  table_ref, asrc_o_ref, adst_o_ref):
    h = jnp.dot(x_ref[...], w1_ref[...], preferred_element_type=jnp.float32)
    hs = h * as1_ref[...]
    hd = h * ad1_ref[...]
    z6 = jnp.zeros((B, 6), jnp.float32)
    for s in range(2):
        a_src = jnp.sum(hs[:, 8 * s:8 * s + 8], axis=1, keepdims=True)
        a_dst = jnp.sum(hd[:, 8 * s:8 * s + 8], axis=1, keepdims=True)
        table_ref[s] = jnp.concatenate(
            [h[:, 8 * s:8 * s + 8], a_src, a_dst, z6], axis=1)
        asrc_o_ref[:, s:s + 1] = a_src
        adst_o_ref[:, s:s + 1] = a_dst


def _stage_a(xpad, W1, as1, ad1):
    return pl.pallas_call(
        _stage_a_body,
        grid=(NPAD // B,),
        in_specs=[
            pl.BlockSpec((B, F_IN), lambda i: (i, 0)),
            pl.BlockSpec((F_IN, 16), lambda i: (0, 0)),
            pl.BlockSpec((1, 16), lambda i: (0, 0)),
            pl.BlockSpec((1, 16), lambda i: (0, 0)),
        ],
        out_specs=[
            pl.BlockSpec((2, B, 16), lambda i: (0, i, 0)),
            pl.BlockSpec((B, 2), lambda i: (i, 0)),
            pl.BlockSpec((B, 2), lambda i: (i, 0)),
        ],
        out_shape=[
            jax.ShapeDtypeStruct((2, NPAD, 16), jnp.float32),
            jax.ShapeDtypeStruct((NPAD, 2), jnp.float32),
            jax.ShapeDtypeStruct((NPAD, 2), jnp.float32),
        ],
    )(xpad, W1, as1, ad1)


def _emit_msg(rows, rowsd, msg, g, iota, nch, acol_d, ngroups=8):
    cold = jnp.full((16,), acol_d, jnp.int32)
    coln = jnp.full((16,), nch, jnp.int32)
    for gk in range(ngroups):
        r = iota + gk * 16
        ad = plsc.load_gather(rowsd, [r, cold])
        asr = plsc.load_gather(rows, [r, coln])
        w = jnp.exp(_lrelu(asr + ad) - g)
        for ch in range(nch):
            cv = jnp.full((16,), ch, jnp.int32)
            col = plsc.load_gather(rows, [r, cv])
            plsc.store_scatter(msg, [r, cv], col * w)
        plsc.store_scatter(msg, [r, coln], w)


L1_STEADY = L1_CHUNKS - 5
L2_FULL = 1562
L2_STEADY = 1560
EPT2 = E // 32


def _sc_l1_body(ei, tab, zeros, gtab, acc_out, acc_sh,
                srcb0, srcb1, dstb0, dstb1, adjd0, adjd1,
                dsts0, dsts1, dsts2, dsts3,
                rows0, rows1, rowsd0, rowsd1,
                msg0, msg1, msg2, msg3, g16,
                se0, se1, sg0, sg1, ss0, ss1, ss2, ss3):
    c = lax.axis_index("c")
    s = lax.axis_index("s")
    pltpu.sync_copy(gtab.at[c], g16)
    for m in (msg0, msg1, msg2, msg3):
        pltpu.sync_copy(zeros.at[pl.ds(0, K), :], m)
    pltpu.sync_copy(zeros.at[pl.ds(s * SLICE, SLICE), :],
                    acc_sh.at[pl.ds(s * SLICE, SLICE), :])
    plsc.subcore_barrier()

    iota = lax.iota(jnp.int32, 16)
    g = g16[...]
    coff = jnp.full((16,), 0, jnp.int32) + c * NPAD
    srcb = (srcb0, srcb1)
    dstb = (dstb0, dstb1)
    adjd = (adjd0, adjd1)
    dsts = (dsts0, dsts1, dsts2, dsts3)
    rows = (rows0, rows1)
    rowsd = (rowsd0, rowsd1)
    msg = (msg0, msg1, msg2, msg3)
    seme = (se0, se1)
    semg = (sg0, sg1)
    sems = (ss0, ss1, ss2, ss3)
    start = s * L1_CHUNKS

    def issue_e(i, sl):
        base = (start + i) * K
        pltpu.async_copy(ei.at[0, pl.ds(base, K)], srcb[sl], seme[sl])
        pltpu.async_copy(ei.at[1, pl.ds(base, K)], dstb[sl], seme[sl])

    def wait_e(sl):
        pltpu.make_async_copy(ei.at[0, pl.ds(0, K)], srcb[sl], seme[sl]).wait()
        pltpu.make_async_copy(ei.at[1, pl.ds(0, K)], dstb[sl], seme[sl]).wait()

    def adjust(sl, sd):
        for gk in range(8):
            d = pl.ds(gk * 16, 16)
            srcb[sl][d] = srcb[sl][d] + coff
            adjd[sl][d] = dstb[sl][d] + coff
            dsts[sd][d] = dstb[sl][d]

    def issue_g(sl):
        pltpu.async_copy(tab.at[srcb[sl]], rows[sl], semg[sl])
        pltpu.async_copy(tab.at[adjd[sl]], rowsd[sl], semg[sl])

    def wait_g(sl):
        pltpu.make_async_copy(tab.at[pl.ds(0, K), :], rows[sl], semg[sl]).wait()
        pltpu.make_async_copy(tab.at[pl.ds(0, K), :], rowsd[sl], semg[sl]).wait()

    def wait_s(sd):
        pltpu.make_async_copy(zeros.at[pl.ds(0, K), :], msg[sd], sems[sd]).wait()

    def piece(i, b, nxt, nxt2, ws):
        se = b % 2
        sn = (b + 1) % 2
        sw = (b + 1) % 4
        if ws:
            wait_s(sw)
        if nxt:
            wait_e(sn)
            adjust(sn, sw)
        wait_g(se)
        if nxt:
            issue_g(sn)
        if nxt2:
            issue_e(i + 2, se)
        _emit_msg(rows[se], rowsd[se], msg[b], g, iota, 8, 9)
        pltpu.async_copy(msg[b], acc_sh.at[dsts[b]], sems[b], add=True)

    issue_e(0, 0)
    issue_e(1, 1)
    wait_e(0)
    adjust(0, 0)
    issue_g(0)
    piece(0, 0, True, True, False)
    piece(1, 1, True, True, False)
    piece(2, 2, True, True, False)
    piece(3, 3, True, True, True)

    Q = L1_STEADY // 4

    @pl.loop(1, Q - 1)
    def _quad(q):
        i0 = 4 * q
        piece(i0, 0, True, True, True)
        piece(i0 + 1, 1, True, True, True)
        piece(i0 + 2, 2, True, True, True)
        piece(i0 + 3, 3, True, True, True)

    i0 = L1_STEADY - 4
    piece(i0, 0, True, True, True)
    piece(i0 + 1, 1, True, True, True)
    piece(i0 + 2, 2, True, False, True)
    piece(i0 + 3, 3, False, False, True)
    wait_s(1)
    wait_s(2)
    wait_s(3)

    for t in range(L1_CHUNKS - L1_STEADY):
        base = (start + L1_STEADY + t) * K
        pltpu.sync_copy(ei.at[0, pl.ds(base, K)], srcb0)
        pltpu.sync_copy(ei.at[1, pl.ds(base, K)], dstb0)
        adjust(0, 0)
        issue_g(0)
        wait_g(0)
        _emit_msg(rows0, rowsd0, msg0, g, iota, 8, 9)
        pltpu.sync_copy(msg0, acc_sh.at[dsts0], add=True)

    plsc.subcore_barrier()
    pltpu.sync_copy(acc_sh.at[pl.ds(s * SLICE, SLICE), :],
                    acc_out.at[c, pl.ds(s * SLICE, SLICE), :])


_sc_l1 = functools.partial(
    pl.kernel,
    out_type=jax.ShapeDtypeStruct((2, NPAD, 16), jnp.float32),
    mesh=_mesh,
    compiler_params=_sc_params,
    scratch_types=(
        [pltpu.VMEM_SHARED((NPAD, 16), jnp.float32)]
        + [pltpu.VMEM((K,), jnp.int32)] * 10
        + [pltpu.VMEM((K, 16), jnp.float32)] * 8
        + [pltpu.VMEM((16,), jnp.float32)]
        + [pltpu.SemaphoreType.DMA] * 8
    ),
)(_sc_l1_body)


def _stage_b_body(acc_ref, tab_ref, asrc_ref, adst_ref, g1_ref,
                  w2_ref, as2_ref, ad2_ref,
                  table2_ref, asrc2_o_ref, adst2_o_ref):
    wself = jnp.exp(_lrelu(asrc_ref[...] + adst_ref[...]) - g1_ref[...])
    outs = []
    for s in range(2):
        h_s = tab_ref[s, :, 0:8]
        ws = wself[:, s:s + 1]
        num = acc_ref[s, :, 0:8] + ws * h_s
        den = acc_ref[s, :, 8:9] + ws
        outs.append(jnp.maximum(num / den, 0.0))
    out1 = jnp.concatenate(outs, axis=1)
    h2 = jnp.dot(out1, w2_ref[...], preferred_element_type=jnp.float32)
    asrc2 = jnp.sum(h2 * as2_ref[...], axis=1, keepdims=True)
    adst2 = jnp.sum(h2 * ad2_ref[...], axis=1, keepdims=True)
    table2_ref[...] = jnp.concatenate(
        [h2, asrc2, adst2, jnp.zeros((B, 11), jnp.float32)], axis=1)
    asrc2_o_ref[...] = asrc2
    adst2_o_ref[...] = adst2


def _stage_b(acc1, table1, asrc1, adst1, g1, W2, as2, ad2):
    return pl.pallas_call(
        _stage_b_body,
        grid=(NPAD // B,),
        in_specs=[
            pl.BlockSpec((2, B, 16), lambda i: (0, i, 0)),
            pl.BlockSpec((2, B, 16), lambda i: (0, i, 0)),
            pl.BlockSpec((B, 2), lambda i: (i, 0)),
            pl.BlockSpec((B, 2), lambda i: (i, 0)),
            pl.BlockSpec((1, 2), lambda i: (0, 0)),
            pl.BlockSpec((16, CLS), lambda i: (0, 0)),
            pl.BlockSpec((1, CLS), lambda i: (0, 0)),
            pl.BlockSpec((1, CLS), lambda i: (0, 0)),
        ],
        out_specs=[
            pl.BlockSpec((B, 16), lambda i: (i, 0)),
            pl.BlockSpec((B, 1), lambda i: (i, 0)),
            pl.BlockSpec((B, 1), lambda i: (i, 0)),
        ],
        out_shape=[
            jax.ShapeDtypeStruct((NPAD, 16), jnp.float32),
            jax.ShapeDtypeStruct((NPAD, 1), jnp.float32),
            jax.ShapeDtypeStruct((NPAD, 1), jnp.float32),
        ],
    )(acc1, table1, asrc1, adst1, g1, W2, as2, ad2)


def _sc_l2_body(ei, tab, zeros, gtab, acc_out, acc_sh,
                srcb0, srcb1, dstb0, dstb1,
                dsts0, dsts1, dsts2, dsts3, srct, dstt,
                rows0, rows1, rowsd0, rowsd1,
                msg0, msg1, msg2, msg3, g16,
                se0, se1, sg0, sg1, ss0, ss1, ss2, ss3):
    c = lax.axis_index("c")
    s = lax.axis_index("s")
    wid = c * 16 + s
    pltpu.sync_copy(gtab.at[c], g16)
    for m in (msg0, msg1, msg2, msg3):
        pltpu.sync_copy(zeros.at[pl.ds(0, K), :], m)
    pltpu.sync_copy(zeros.at[pl.ds(s * SLICE, SLICE), :],
                    acc_sh.at[pl.ds(s * SLICE, SLICE), :])
    plsc.subcore_barrier()

    iota = lax.iota(jnp.int32, 16)
    g = g16[...]
    srcb = (srcb0, srcb1)
    dstb = (dstb0, dstb1)
    dsts = (dsts0, dsts1, dsts2, dsts3)
    rows = (rows0, rows1)
    rowsd = (rowsd0, rowsd1)
    msg = (msg0, msg1, msg2, msg3)
    seme = (se0, se1)
    semg = (sg0, sg1)
    sems = (ss0, ss1, ss2, ss3)
    estart = wid * EPT2

    def issue_e(i, sl):
        base = estart + i * K
        pltpu.async_copy(ei.at[0, pl.ds(base, K)], srcb[sl], seme[sl])
        pltpu.async_copy(ei.at[1, pl.ds(base, K)], dstb[sl], seme[sl])

    def wait_e(sl):
        pltpu.make_async_copy(ei.at[0, pl.ds(0, K)], srcb[sl], seme[sl]).wait()
        pltpu.make_async_copy(ei.at[1, pl.ds(0, K)], dstb[sl], seme[sl]).wait()

    def adjust(sl, sd):
        for gk in range(8):
            d = pl.ds(gk * 16, 16)
            dsts[sd][d] = dstb[sl][d]

    def issue_g(sl):
        pltpu.async_copy(tab.at[srcb[sl]], rows[sl], semg[sl])
        pltpu.async_copy(tab.at[dstb[sl]], rowsd[sl], semg[sl])

    def wait_g(sl):
        pltpu.make_async_copy(tab.at[pl.ds(0, K), :], rows[sl], semg[sl]).wait()
        pltpu.make_async_copy(tab.at[pl.ds(0, K), :], rowsd[sl], semg[sl]).wait()

    def wait_s(sd):
        pltpu.make_async_copy(zeros.at[pl.ds(0, K), :], msg[sd], sems[sd]).wait()

    def piece(i, b, nxt, nxt2, ws):
        se = b % 2
        sn = (b + 1) % 2
        sw = (b + 1) % 4
        if ws:
            wait_s(sw)
        if nxt:
            wait_e(sn)
            adjust(sn, sw)
        wait_g(se)
        if nxt:
            issue_g(sn)
        if nxt2:
            issue_e(i + 2, se)
        _emit_msg(rows[se], rowsd[se], msg[b], g, iota, CLS, CLS + 1)
        pltpu.async_copy(msg[b], acc_sh.at[dsts[b]], sems[b], add=True)

    issue_e(0, 0)
    issue_e(1, 1)
    wait_e(0)
    adjust(0, 0)
    issue_g(0)
    piece(0, 0, True, True, False)
    piece(1, 1, True, True, False)
    piece(2, 2, True, True, False)
    piece(3, 3, True, True, True)

    Q = L2_STEADY // 4

    @pl.loop(1, Q - 1)
    def _quad(q):
        i0 = 4 * q
        piece(i0, 0, True, True, True)
        piece(i0 + 1, 1, True, True, True)
        piece(i0 + 2, 2, True, True, True)
        piece(i0 + 3, 3, True, True, True)

    i0 = L2_STEADY - 4
    piece(i0, 0, True, True, True)
    piece(i0 + 1, 1, True, True, True)
    piece(i0 + 2, 2, True, False, True)
    piece(i0 + 3, 3, False, False, True)
    wait_s(1)
    wait_s(2)
    wait_s(3)

    for t in range(L2_FULL - L2_STEADY):
        base = estart + (L2_STEADY + t) * K
        pltpu.sync_copy(ei.at[0, pl.ds(base, K)], srcb0)
        pltpu.sync_copy(ei.at[1, pl.ds(base, K)], dstb0)
        adjust(0, 0)
        issue_g(0)
        wait_g(0)
        _emit_msg(rows0, rowsd0, msg0, g, iota, CLS, CLS + 1)
        pltpu.sync_copy(msg0, acc_sh.at[dsts0], add=True)

    mb = estart + L2_FULL * K
    pltpu.sync_copy(ei.at[0, pl.ds(mb, 64)], srct)
    pltpu.sync_copy(ei.at[1, pl.ds(mb, 64)], dstt)
    d1 = pltpu.async_copy(tab.at[srct], rows0.at[pl.ds(0, 64), :], sg0)
    d2 = pltpu.async_copy(tab.at[dstt], rowsd0.at[pl.ds(0, 64), :], sg0)
    d1.wait()
    d2.wait()
    _emit_msg(rows0, rowsd0, msg0, g, iota, CLS, CLS + 1, ngroups=4)
    pltpu.sync_copy(msg0.at[pl.ds(0, 64), :], acc_sh.at[dstt], add=True)

    plsc.subcore_barrier()
    pltpu.sync_copy(acc_sh.at[pl.ds(s * SLICE, SLICE), :],
                    acc_out.at[c, pl.ds(s * SLICE, SLICE), :])


_sc_l2 = functools.partial(
    pl.kernel,
    out_type=jax.ShapeDtypeStruct((2, NPAD, 16), jnp.float32),
    mesh=_mesh,
    compiler_params=_sc_params,
    scratch_types=(
        [pltpu.VMEM_SHARED((NPAD, 16), jnp.float32)]
        + [pltpu.VMEM((K,), jnp.int32)] * 8
        + [pltpu.VMEM((64,), jnp.int32)] * 2
        + [pltpu.VMEM((K, 16), jnp.float32)] * 8
        + [pltpu.VMEM((16,), jnp.float32)]
        + [pltpu.SemaphoreType.DMA] * 8
    ),
)(_sc_l2_body)


def _stage_c_body(acc_ref, tab2_ref, g2_ref, o_ref):
    h2 = tab2_ref[:, 0:CLS]
    asrc2 = tab2_ref[:, CLS:CLS + 1]
    adst2 = tab2_ref[:, CLS + 1:CLS + 2]
    wself = jnp.exp(_lrelu(asrc2 + adst2) - g2_ref[...])
    num = acc_ref[0, :, 0:CLS] + acc_ref[1, :, 0:CLS] + wself * h2
    den = acc_ref[0, :, CLS:CLS + 1] + acc_ref[1, :, CLS:CLS + 1] + wself
    h = num / den
    m = jnp.max(h, axis=-1, keepdims=True)
    ex = jnp.exp(h - m)
    o_ref[...] = ex / jnp.sum(ex, axis=-1, keepdims=True)


def _stage_c(acc2, table2, g2):
    return pl.pallas_call(
        _stage_c_body,
        grid=(NPAD // B,),
        in_specs=[
            pl.BlockSpec((2, B, 16), lambda i: (0, i, 0)),
            pl.BlockSpec((B, 16), lambda i: (i, 0)),
            pl.BlockSpec((1, 1), lambda i: (0, 0)),
        ],
        out_specs=pl.BlockSpec((B, CLS), lambda i: (i, 0)),
        out_shape=jax.ShapeDtypeStruct((NPAD, CLS), jnp.float32),
    )(acc2, table2, g2)


def kernel(x, edge_index, W1, att_src1, att_dst1, W2, att_src2, att_dst2):
    xpad = jnp.pad(x, ((0, NPAD - N), (0, 0)))
    as1 = att_src1.reshape(1, 16)
    ad1 = att_dst1.reshape(1, 16)
    table1, asrc1, adst1 = _stage_a(xpad, W1, as1, ad1)

    g1 = _lrelu(jnp.max(asrc1, axis=0) + jnp.max(adst1, axis=0))
    gtab1 = jnp.tile(g1[:, None], (1, 16))
    zeros16 = jnp.zeros((NPAD, 16), jnp.float32)

    acc1 = _sc_l1(edge_index, table1.reshape(2 * NPAD, 16), zeros16, gtab1)

    table2, asrc2, adst2 = _stage_b(
        acc1, table1, asrc1, adst1, g1.reshape(1, 2),
        W2, att_src2.reshape(1, CLS), att_dst2.reshape(1, CLS))

    g2 = _lrelu(jnp.max(asrc2) + jnp.max(adst2))
    gtab2 = jnp.full((2, 16), g2, jnp.float32)

    acc2 = _sc_l2(edge_index, table2, zeros16, gtab2)

    out = _stage_c(acc2, table2, g2.reshape(1, 1))
    return out[:N]


# speedup vs baseline: 241.6726x; 1.2705x over previous
import functools
import jax
import jax.numpy as jnp
from jax import lax
from jax.experimental import pallas as pl
from jax.experimental.pallas import tpu as pltpu
from jax.experimental.pallas import tpu_sc as plsc

N = 100000
E = 6400000
F_IN = 4
HID = 8
HEADS = 2
CLS = 3
SLOPE = 0.2

NPAD = 102400
SLICE = NPAD // 16
B = 1024
K = 128
NCHUNK = E // K
L1_CHUNKS = NCHUNK // 16

_mesh = plsc.VectorSubcoreMesh(
    core_axis_name="c", subcore_axis_name="s", num_cores=2, num_subcores=16)

_sc_params = pltpu.CompilerParams(
    needs_layout_passes=False, use_tc_tiling_on_sc=False)


def _lrelu(x):
    return jnp.maximum(x, x * SLOPE)


def _stage_a_body(x_ref, w1_ref, as1_ref, ad1_ref,
                  table_ref, asrc_o_ref, adst_o_ref):
    h = jnp.dot(x_ref[...], w1_ref[...], preferred_element_type=jnp.float32)
    hs = h * as1_ref[...]
    hd = h * ad1_ref[...]
    z6 = jnp.zeros((B, 6), jnp.float32)
    for s in range(2):
        a_src = jnp.sum(hs[:, 8 * s:8 * s + 8], axis=1, keepdims=True)
        a_dst = jnp.sum(hd[:, 8 * s:8 * s + 8], axis=1, keepdims=True)
        table_ref[s] = jnp.concatenate(
            [h[:, 8 * s:8 * s + 8], a_src, a_dst, z6], axis=1)
        asrc_o_ref[:, s:s + 1] = a_src
        adst_o_ref[:, s:s + 1] = a_dst


def _stage_a(xpad, W1, as1, ad1):
    return pl.pallas_call(
        _stage_a_body,
        grid=(NPAD // B,),
        in_specs=[
            pl.BlockSpec((B, F_IN), lambda i: (i, 0)),
            pl.BlockSpec((F_IN, 16), lambda i: (0, 0)),
            pl.BlockSpec((1, 16), lambda i: (0, 0)),
            pl.BlockSpec((1, 16), lambda i: (0, 0)),
        ],
        out_specs=[
            pl.BlockSpec((2, B, 16), lambda i: (0, i, 0)),
            pl.BlockSpec((B, 2), lambda i: (i, 0)),
            pl.BlockSpec((B, 2), lambda i: (i, 0)),
        ],
        out_shape=[
            jax.ShapeDtypeStruct((2, NPAD, 16), jnp.float32),
            jax.ShapeDtypeStruct((NPAD, 2), jnp.float32),
            jax.ShapeDtypeStruct((NPAD, 2), jnp.float32),
        ],
    )(xpad, W1, as1, ad1)


def _emit_msg(rows, rowsd, msg, g, iota, nch, acol_d, ngroups=8):
    cold = jnp.full((16,), acol_d, jnp.int32)
    coln = jnp.full((16,), nch, jnp.int32)
    for gk in range(ngroups):
        r = iota + gk * 16
        ad = plsc.load_gather(rowsd, [r, cold])
        asr = plsc.load_gather(rows, [r, coln])
        w = jnp.exp(_lrelu(asr + ad) - g)
        for ch in range(nch):
            cv = jnp.full((16,), ch, jnp.int32)
            col = plsc.load_gather(rows, [r, cv])
            plsc.store_scatter(msg, [r, cv], col * w)
        plsc.store_scatter(msg, [r, coln], w)


L1_STEADY = L1_CHUNKS - 5
L2_FULL = 1562
L2_STEADY = 1560
EPT2 = E // 32


def _sc_l1_body(ei, tab, zeros, gtab, acc_out, acc_sh,
                srcb0, srcb1, srcb2, srcb3, dstb0, dstb1, dstb2, dstb3,
                adjd0, adjd1, adjd2, adjd3, dsts0, dsts1, dsts2, dsts3,
                rows0, rows1, rows2, rows3, rowsd0, rowsd1, rowsd2, rowsd3,
                msg0, msg1, msg2, msg3, g16,
                se0, se1, se2, se3, sg0, sg1, sg2, sg3, ss0, ss1, ss2, ss3):
    c = lax.axis_index("c")
    s = lax.axis_index("s")
    pltpu.sync_copy(gtab.at[c], g16)
    for m in (msg0, msg1, msg2, msg3):
        pltpu.sync_copy(zeros.at[pl.ds(0, K), :], m)
    pltpu.sync_copy(zeros.at[pl.ds(s * SLICE, SLICE), :],
                    acc_sh.at[pl.ds(s * SLICE, SLICE), :])
    plsc.subcore_barrier()

    iota = lax.iota(jnp.int32, 16)
    g = g16[...]
    coff = jnp.full((16,), 0, jnp.int32) + c * NPAD
    srcb = (srcb0, srcb1, srcb2, srcb3)
    dstb = (dstb0, dstb1, dstb2, dstb3)
    adjd = (adjd0, adjd1, adjd2, adjd3)
    dsts = (dsts0, dsts1, dsts2, dsts3)
    rows = (rows0, rows1, rows2, rows3)
    rowsd = (rowsd0, rowsd1, rowsd2, rowsd3)
    msg = (msg0, msg1, msg2, msg3)
    seme = (se0, se1, se2, se3)
    semg = (sg0, sg1, sg2, sg3)
    sems = (ss0, ss1, ss2, ss3)
    start = s * L1_CHUNKS

    def issue_e(i, sl):
        base = (start + i) * K
        pltpu.async_copy(ei.at[0, pl.ds(base, K)], srcb[sl], seme[sl])
        pltpu.async_copy(ei.at[1, pl.ds(base, K)], dstb[sl], seme[sl])

    def wait_e(sl):
        pltpu.make_async_copy(ei.at[0, pl.ds(0, K)], srcb[sl], seme[sl]).wait()
        pltpu.make_async_copy(ei.at[1, pl.ds(0, K)], dstb[sl], seme[sl]).wait()

    def adjust_idx(sl):
        for gk in range(8):
            d = pl.ds(gk * 16, 16)
            srcb[sl][d] = srcb[sl][d] + coff
            adjd[sl][d] = dstb[sl][d] + coff

    def copy_dsts(sl):
        for gk in range(8):
            d = pl.ds(gk * 16, 16)
            dsts[sl][d] = dstb[sl][d]

    def issue_g(sl):
        pltpu.async_copy(tab.at[srcb[sl]], rows[sl], semg[sl])
        pltpu.async_copy(tab.at[adjd[sl]], rowsd[sl], semg[sl])

    def wait_g(sl):
        pltpu.make_async_copy(tab.at[pl.ds(0, K), :], rows[sl], semg[sl]).wait()
        pltpu.make_async_copy(tab.at[pl.ds(0, K), :], rowsd[sl], semg[sl]).wait()

    def wait_s(sd):
        pltpu.make_async_copy(zeros.at[pl.ds(0, K), :], msg[sd], sems[sd]).wait()

    def piece(i, b, ws, f_d, f_g, f_e2):
        if ws:
            wait_s((b + 1) % 4)
        if f_d:
            copy_dsts((b + 1) % 4)
        if f_g:
            wait_e((b + 3) % 4)
            adjust_idx((b + 3) % 4)
        wait_g(b)
        if f_g:
            issue_g((b + 3) % 4)
        if f_e2:
            issue_e(i + 4, b)
        _emit_msg(rows[b], rowsd[b], msg[b], g, iota, 8, 9)
        pltpu.async_copy(msg[b], acc_sh.at[dsts[b]], sems[b], add=True)

    for j in range(4):
        issue_e(j, j)
    for j in range(3):
        wait_e(j)
        adjust_idx(j)
        issue_g(j)
    copy_dsts(0)
    piece(0, 0, False, True, True, True)
    piece(1, 1, False, True, True, True)
    piece(2, 2, False, True, True, True)
    piece(3, 3, True, True, True, True)

    Q = L1_STEADY // 4

    @pl.loop(1, Q - 1)
    def _quad(q):
        i0 = 4 * q
        piece(i0, 0, True, True, True, True)
        piece(i0 + 1, 1, True, True, True, True)
        piece(i0 + 2, 2, True, True, True, True)
        piece(i0 + 3, 3, True, True, True, True)

    i0 = L1_STEADY - 4
    piece(i0, 0, True, True, True, False)
    piece(i0 + 1, 1, True, True, False, False)
    piece(i0 + 2, 2, True, True, False, False)
    piece(i0 + 3, 3, True, False, False, False)
    wait_s(1)
    wait_s(2)
    wait_s(3)

    for t in range(L1_CHUNKS - L1_STEADY):
        base = (start + L1_STEADY + t) * K
        pltpu.sync_copy(ei.at[0, pl.ds(base, K)], srcb0)
        pltpu.sync_copy(ei.at[1, pl.ds(base, K)], dstb0)
        adjust_idx(0)
        copy_dsts(0)
        issue_g(0)
        wait_g(0)
        _emit_msg(rows0, rowsd0, msg0, g, iota, 8, 9)
        pltpu.sync_copy(msg0, acc_sh.at[dsts0], add=True)

    plsc.subcore_barrier()
    pltpu.sync_copy(acc_sh.at[pl.ds(s * SLICE, SLICE), :],
                    acc_out.at[c, pl.ds(s * SLICE, SLICE), :])


_sc_l1 = functools.partial(
    pl.kernel,
    out_type=jax.ShapeDtypeStruct((2, NPAD, 16), jnp.float32),
    mesh=_mesh,
    compiler_params=_sc_params,
    scratch_types=(
        [pltpu.VMEM_SHARED((NPAD, 16), jnp.float32)]
        + [pltpu.VMEM((K,), jnp.int32)] * 16
        + [pltpu.VMEM((K, 16), jnp.float32)] * 12
        + [pltpu.VMEM((16,), jnp.float32)]
        + [pltpu.SemaphoreType.DMA] * 12
    ),
)(_sc_l1_body)


def _stage_b_body(acc_ref, tab_ref, asrc_ref, adst_ref, g1_ref,
                  w2_ref, as2_ref, ad2_ref,
                  table2_ref, asrc2_o_ref, adst2_o_ref):
    wself = jnp.exp(_lrelu(asrc_ref[...] + adst_ref[...]) - g1_ref[...])
    outs = []
    for s in range(2):
        h_s = tab_ref[s, :, 0:8]
        ws = wself[:, s:s + 1]
        num = acc_ref[s, :, 0:8] + ws * h_s
        den = acc_ref[s, :, 8:9] + ws
        outs.append(jnp.maximum(num / den, 0.0))
    out1 = jnp.concatenate(outs, axis=1)
    h2 = jnp.dot(out1, w2_ref[...], preferred_element_type=jnp.float32)
    asrc2 = jnp.sum(h2 * as2_ref[...], axis=1, keepdims=True)
    adst2 = jnp.sum(h2 * ad2_ref[...], axis=1, keepdims=True)
    table2_ref[...] = jnp.concatenate(
        [h2, asrc2, adst2, jnp.zeros((B, 11), jnp.float32)], axis=1)
    asrc2_o_ref[...] = asrc2
    adst2_o_ref[...] = adst2


def _stage_b(acc1, table1, asrc1, adst1, g1, W2, as2, ad2):
    return pl.pallas_call(
        _stage_b_body,
        grid=(NPAD // B,),
        in_specs=[
            pl.BlockSpec((2, B, 16), lambda i: (0, i, 0)),
            pl.BlockSpec((2, B, 16), lambda i: (0, i, 0)),
            pl.BlockSpec((B, 2), lambda i: (i, 0)),
            pl.BlockSpec((B, 2), lambda i: (i, 0)),
            pl.BlockSpec((1, 2), lambda i: (0, 0)),
            pl.BlockSpec((16, CLS), lambda i: (0, 0)),
            pl.BlockSpec((1, CLS), lambda i: (0, 0)),
            pl.BlockSpec((1, CLS), lambda i: (0, 0)),
        ],
        out_specs=[
            pl.BlockSpec((B, 16), lambda i: (i, 0)),
            pl.BlockSpec((B, 1), lambda i: (i, 0)),
            pl.BlockSpec((B, 1), lambda i: (i, 0)),
        ],
        out_shape=[
            jax.ShapeDtypeStruct((NPAD, 16), jnp.float32),
            jax.ShapeDtypeStruct((NPAD, 1), jnp.float32),
            jax.ShapeDtypeStruct((NPAD, 1), jnp.float32),
        ],
    )(acc1, table1, asrc1, adst1, g1, W2, as2, ad2)


def _sc_l2_body(ei, tab, zeros, gtab, acc_out, acc_sh,
                srcb0, srcb1, srcb2, srcb3, dstb0, dstb1, dstb2, dstb3,
                dsts0, dsts1, dsts2, dsts3, srct, dstt,
                rows0, rows1, rows2, rows3, rowsd0, rowsd1, rowsd2, rowsd3,
                msg0, msg1, msg2, msg3, g16,
                se0, se1, se2, se3, sg0, sg1, sg2, sg3, ss0, ss1, ss2, ss3):
    c = lax.axis_index("c")
    s = lax.axis_index("s")
    wid = c * 16 + s
    pltpu.sync_copy(gtab.at[c], g16)
    for m in (msg0, msg1, msg2, msg3):
        pltpu.sync_copy(zeros.at[pl.ds(0, K), :], m)
    pltpu.sync_copy(zeros.at[pl.ds(s * SLICE, SLICE), :],
                    acc_sh.at[pl.ds(s * SLICE, SLICE), :])
    plsc.subcore_barrier()

    iota = lax.iota(jnp.int32, 16)
    g = g16[...]
    srcb = (srcb0, srcb1, srcb2, srcb3)
    dstb = (dstb0, dstb1, dstb2, dstb3)
    dsts = (dsts0, dsts1, dsts2, dsts3)
    rows = (rows0, rows1, rows2, rows3)
    rowsd = (rowsd0, rowsd1, rowsd2, rowsd3)
    msg = (msg0, msg1, msg2, msg3)
    seme = (se0, se1, se2, se3)
    semg = (sg0, sg1, sg2, sg3)
    sems = (ss0, ss1, ss2, ss3)
    estart = wid * EPT2

    def issue_e(i, sl):
        base = estart + i * K
        pltpu.async_copy(ei.at[0, pl.ds(base, K)], srcb[sl], seme[sl])
        pltpu.async_copy(ei.at[1, pl.ds(base, K)], dstb[sl], seme[sl])

    def wait_e(sl):
        pltpu.make_async_copy(ei.at[0, pl.ds(0, K)], srcb[sl], seme[sl]).wait()
        pltpu.make_async_copy(ei.at[1, pl.ds(0, K)], dstb[sl], seme[sl]).wait()

    def copy_dsts(sl):
        for gk in range(8):
            d = pl.ds(gk * 16, 16)
            dsts[sl][d] = dstb[sl][d]

    def issue_g(sl):
        pltpu.async_copy(tab.at[srcb[sl]], rows[sl], semg[sl])
        pltpu.async_copy(tab.at[dstb[sl]], rowsd[sl], semg[sl])

    def wait_g(sl):
        pltpu.make_async_copy(tab.at[pl.ds(0, K), :], rows[sl], semg[sl]).wait()
        pltpu.make_async_copy(tab.at[pl.ds(0, K), :], rowsd[sl], semg[sl]).wait()

    def wait_s(sd):
        pltpu.make_async_copy(zeros.at[pl.ds(0, K), :], msg[sd], sems[sd]).wait()

    def piece(i, b, ws, f_d, f_g, f_e2):
        if ws:
            wait_s((b + 1) % 4)
        if f_d:
            copy_dsts((b + 1) % 4)
        if f_g:
            wait_e((b + 3) % 4)
        wait_g(b)
        if f_g:
            issue_g((b + 3) % 4)
        if f_e2:
            issue_e(i + 4, b)
        _emit_msg(rows[b], rowsd[b], msg[b], g, iota, CLS, CLS + 1)
        pltpu.async_copy(msg[b], acc_sh.at[dsts[b]], sems[b], add=True)

    for j in range(4):
        issue_e(j, j)
    for j in range(3):
        wait_e(j)
        issue_g(j)
    copy_dsts(0)
    piece(0, 0, False, True, True, True)
    piece(1, 1, False, True, True, True)
    piece(2, 2, False, True, True, True)
    piece(3, 3, True, True, True, True)

    Q = L2_STEADY // 4

    @pl.loop(1, Q - 1)
    def _quad(q):
        i0 = 4 * q
        piece(i0, 0, True, True, True, True)
        piece(i0 + 1, 1, True, True, True, True)
        piece(i0 + 2, 2, True, True, True, True)
        piece(i0 + 3, 3, True, True, True, True)

    i0 = L2_STEADY - 4
    piece(i0, 0, True, True, True, False)
    piece(i0 + 1, 1, True, True, False, False)
    piece(i0 + 2, 2, True, True, False, False)
    piece(i0 + 3, 3, True, False, False, False)
    wait_s(1)
    wait_s(2)
    wait_s(3)

    for t in range(L2_FULL - L2_STEADY):
        base = estart + (L2_STEADY + t) * K
        pltpu.sync_copy(ei.at[0, pl.ds(base, K)], srcb0)
        pltpu.sync_copy(ei.at[1, pl.ds(base, K)], dstb0)
        copy_dsts(0)
        issue_g(0)
        wait_g(0)
        _emit_msg(rows0, rowsd0, msg0, g, iota, CLS, CLS + 1)
        pltpu.sync_copy(msg0, acc_sh.at[dsts0], add=True)

    mb = estart + L2_FULL * K
    pltpu.sync_copy(ei.at[0, pl.ds(mb, 64)], srct)
    pltpu.sync_copy(ei.at[1, pl.ds(mb, 64)], dstt)
    d1 = pltpu.async_copy(tab.at[srct], rows0.at[pl.ds(0, 64), :], sg0)
    d2 = pltpu.async_copy(tab.at[dstt], rowsd0.at[pl.ds(0, 64), :], sg0)
    d1.wait()
    d2.wait()
    _emit_msg(rows0, rowsd0, msg0, g, iota, CLS, CLS + 1, ngroups=4)
    pltpu.sync_copy(msg0.at[pl.ds(0, 64), :], acc_sh.at[dstt], add=True)

    plsc.subcore_barrier()
    pltpu.sync_copy(acc_sh.at[pl.ds(s * SLICE, SLICE), :],
                    acc_out.at[c, pl.ds(s * SLICE, SLICE), :])


_sc_l2 = functools.partial(
    pl.kernel,
    out_type=jax.ShapeDtypeStruct((2, NPAD, 16), jnp.float32),
    mesh=_mesh,
    compiler_params=_sc_params,
    scratch_types=(
        [pltpu.VMEM_SHARED((NPAD, 16), jnp.float32)]
        + [pltpu.VMEM((K,), jnp.int32)] * 12
        + [pltpu.VMEM((64,), jnp.int32)] * 2
        + [pltpu.VMEM((K, 16), jnp.float32)] * 12
        + [pltpu.VMEM((16,), jnp.float32)]
        + [pltpu.SemaphoreType.DMA] * 12
    ),
)(_sc_l2_body)


def _stage_c_body(acc_ref, tab2_ref, g2_ref, o_ref):
    h2 = tab2_ref[:, 0:CLS]
    asrc2 = tab2_ref[:, CLS:CLS + 1]
    adst2 = tab2_ref[:, CLS + 1:CLS + 2]
    wself = jnp.exp(_lrelu(asrc2 + adst2) - g2_ref[...])
    num = acc_ref[0, :, 0:CLS] + acc_ref[1, :, 0:CLS] + wself * h2
    den = acc_ref[0, :, CLS:CLS + 1] + acc_ref[1, :, CLS:CLS + 1] + wself
    h = num / den
    m = jnp.max(h, axis=-1, keepdims=True)
    ex = jnp.exp(h - m)
    o_ref[...] = ex / jnp.sum(ex, axis=-1, keepdims=True)


def _stage_c(acc2, table2, g2):
    return pl.pallas_call(
        _stage_c_body,
        grid=(NPAD // B,),
        in_specs=[
            pl.BlockSpec((2, B, 16), lambda i: (0, i, 0)),
            pl.BlockSpec((B, 16), lambda i: (i, 0)),
            pl.BlockSpec((1, 1), lambda i: (0, 0)),
        ],
        out_specs=pl.BlockSpec((B, CLS), lambda i: (i, 0)),
        out_shape=jax.ShapeDtypeStruct((NPAD, CLS), jnp.float32),
    )(acc2, table2, g2)


def kernel(x, edge_index, W1, att_src1, att_dst1, W2, att_src2, att_dst2):
    xpad = jnp.pad(x, ((0, NPAD - N), (0, 0)))
    as1 = att_src1.reshape(1, 16)
    ad1 = att_dst1.reshape(1, 16)
    table1, asrc1, adst1 = _stage_a(xpad, W1, as1, ad1)

    g1 = _lrelu(jnp.max(asrc1, axis=0) + jnp.max(adst1, axis=0))
    gtab1 = jnp.tile(g1[:, None], (1, 16))
    zeros16 = jnp.zeros((NPAD, 16), jnp.float32)

    acc1 = _sc_l1(edge_index, table1.reshape(2 * NPAD, 16), zeros16, gtab1)

    table2, asrc2, adst2 = _stage_b(
        acc1, table1, asrc1, adst1, g1.reshape(1, 2),
        W2, att_src2.reshape(1, CLS), att_dst2.reshape(1, CLS))

    g2 = _lrelu(jnp.max(asrc2) + jnp.max(adst2))
    gtab2 = jnp.full((2, 16), g2, jnp.float32)

    acc2 = _sc_l2(edge_index, table2, zeros16, gtab2)

    out = _stage_c(acc2, table2, g2.reshape(1, 1))
    return out[:N]
